# baseline (speedup 1.0000x reference)
import functools
import jax
import jax.numpy as jnp
from jax import lax
from jax.experimental import pallas as pl
from jax.experimental.pallas import tpu as pltpu
from jax.experimental.pallas import tpu_sc as plsc

DIM = 128
H = 4
HD = 32
BSZ = 1024
ACC_R = BSZ + 8
N_ROWS = 100000
N_H = N_ROWS // 2
PAD_H = 51200
BLK = 400
NBLK_H = N_H // BLK
NC = 2
NS = 16
NW = NC * NS
ROWS_W = PAD_H // NW
CH = 32
NCH = ROWS_W // CH
ROWS_T = BSZ // NS


def _bmat16():
    hrow = lax.broadcasted_iota(jnp.int32, (16, DIM), 0)
    hcol = lax.broadcasted_iota(jnp.int32, (16, DIM), 1) // HD
    return (hrow == hcol).astype(jnp.float32)


def _proj_body(batch_ref, x_ref, wvt_ref, a_ref, c_ref, bv_ref,
               ev_ref, s_ref, accs):
    i = pl.program_id(0)

    @pl.when(i == 0)
    def _init():
        accs[...] = jnp.zeros_like(accs)

    x = x_ref[...]
    v = jnp.dot(x, wvt_ref[...],
                preferred_element_type=jnp.float32) + bv_ref[...]
    attn = jnp.dot(x, a_ref[...],
                   preferred_element_type=jnp.float32) + c_ref[...]
    e = jnp.exp(attn)
    eb = jnp.dot(e, _bmat16(), preferred_element_type=jnp.float32)
    ev_ref[...] = eb * v

    brow = batch_ref[0]
    seg = lax.broadcasted_iota(jnp.int32, (BSZ, BLK), 0).astype(jnp.float32)
    oht = (jnp.broadcast_to(brow, (BSZ, BLK)) == seg).astype(jnp.bfloat16)
    accs[...] += jnp.dot(oht, e.astype(jnp.bfloat16),
                         preferred_element_type=jnp.float32)

    @pl.when(i == pl.num_programs(0) - 1)
    def _fin():
        s_ref[...] = accs[...]


def _stage1(x, batchf, half):
    off = half * NBLK_H
    return pl.pallas_call(
        _proj_body,
        grid=(NBLK_H,),
        in_specs=[
            pl.BlockSpec((1, 1, BLK), lambda i: (i + off, 0, 0)),
            pl.BlockSpec((BLK, DIM), lambda i: (i + off, 0)),
            pl.BlockSpec((DIM, DIM), lambda i: (0, 0)),
            pl.BlockSpec((DIM, 16), lambda i: (0, 0)),
            pl.BlockSpec((1, 16), lambda i: (0, 0)),
            pl.BlockSpec((1, DIM), lambda i: (0, 0)),
        ],
        out_specs=[
            pl.BlockSpec((BLK, DIM), lambda i: (i, 0)),
            pl.BlockSpec((BSZ, 16), lambda i: (0, 0)),
        ],
        out_shape=[
            jax.ShapeDtypeStruct((PAD_H, DIM), jnp.float32),
            jax.ShapeDtypeStruct((BSZ, 16), jnp.float32),
        ],
        scratch_shapes=[pltpu.VMEM((BSZ, 16), jnp.float32)],
    )


def _sc_body(ev_h, idx_h, zp_h, pout_h, ibuf, vbuf_a, vbuf_b,
             accp, sem_a, sem_b):
    cid = lax.axis_index("c")
    sid = lax.axis_index("s")
    wid = cid * NS + sid

    @pl.when(sid == 0)
    def _init():
        pltpu.sync_copy(zp_h, accp)
    plsc.subcore_barrier()

    pltpu.sync_copy(idx_h.at[wid], ibuf)
    base = wid * ROWS_W

    def start_load(j, buf, sem):
        pltpu.async_copy(ev_h.at[pl.ds(base + j * CH, CH)], buf, sem)

    def wait_load(buf, sem):
        pltpu.make_async_copy(ev_h.at[pl.ds(0, CH)], buf, sem).wait()

    start_load(0, vbuf_a, sem_a)

    def outer(t, carry):
        jj = 2 * t
        start_load(jj + 1, vbuf_b, sem_b)
        wait_load(vbuf_a, sem_a)
        pltpu.sync_copy(vbuf_a, accp.at[ibuf.at[jj]], add=True)

        @pl.when(jj + 2 < NCH)
        def _():
            start_load(jj + 2, vbuf_a, sem_a)
        wait_load(vbuf_b, sem_b)
        pltpu.sync_copy(vbuf_b, accp.at[ibuf.at[jj + 1]], add=True)
        return carry

    lax.fori_loop(0, NCH // 2, outer, 0)
    plsc.subcore_barrier()

    r0 = sid * ROWS_T
    pltpu.sync_copy(accp.at[pl.ds(r0, ROWS_T)],
                    pout_h.at[cid, pl.ds(r0, ROWS_T)])


def _sc_call(ev, idx3):
    zp = jnp.zeros((ACC_R, DIM), jnp.float32)
    mesh = plsc.VectorSubcoreMesh(core_axis_name="c", subcore_axis_name="s")
    sc_fn = functools.partial(
        pl.kernel,
        out_type=jax.ShapeDtypeStruct((NC, BSZ, DIM), jnp.float32),
        mesh=mesh,
        scratch_types=[
            pltpu.VMEM((NCH, CH), jnp.int32),
            pltpu.VMEM((CH, DIM), jnp.float32),
            pltpu.VMEM((CH, DIM), jnp.float32),
            pltpu.VMEM_SHARED((ACC_R, DIM), jnp.float32),
            pltpu.SemaphoreType.DMA,
            pltpu.SemaphoreType.DMA,
        ],
    )(_sc_body)
    return sc_fn(ev, idx3, zp)


def _comb_body(pa_ref, pb_ref, sa_ref, sb_ref, out_ref):
    p = pa_ref[0] + pa_ref[1] + pb_ref[0] + pb_ref[1]
    s = sa_ref[...] + sb_ref[...]
    sb = jnp.dot(s, _bmat16(), preferred_element_type=jnp.float32)
    out_ref[...] = p / (sb + 1e-8)


def kernel(x, batch, query, W_k, b_k, W_v, b_v):
    scale = HD ** -0.5
    wkt = W_k.T
    a4 = scale * (wkt.reshape(DIM, H, HD) * query[None, :, :]).sum(-1)
    a16 = jnp.pad(a4, ((0, 0), (0, 12)))
    c4 = scale * (b_k.reshape(H, HD) * query).sum(-1)
    c16 = jnp.pad(c4, (0, 12)).reshape(1, 16)
    wvt = W_v.T
    bv = b_v.reshape(1, DIM)

    bi = batch.astype(jnp.int32)
    batchf = bi.astype(jnp.float32).reshape(2 * NBLK_H, 1, BLK)
    idx_a = jnp.pad(bi[:N_H], (0, PAD_H - N_H),
                    constant_values=BSZ).reshape(NW, NCH, CH)
    idx_b = jnp.pad(bi[N_H:], (0, PAD_H - N_H),
                    constant_values=BSZ).reshape(NW, NCH, CH)

    ev_a, s_a = _stage1(x, batchf, 0)(batchf, x, wvt, a16, c16, bv)
    p_a = _sc_call(ev_a, idx_a)
    ev_b, s_b = _stage1(x, batchf, 1)(batchf, x, wvt, a16, c16, bv)
    p_b = _sc_call(ev_b, idx_b)

    out = pl.pallas_call(
        _comb_body,
        out_shape=jax.ShapeDtypeStruct((BSZ, DIM), jnp.float32),
    )(p_a, p_b, s_a, s_b)
    return out


# speedup vs baseline: 1.4043x; 1.4043x over previous
import functools
import jax
import jax.numpy as jnp
from jax import lax
from jax.experimental import pallas as pl
from jax.experimental.pallas import tpu as pltpu
from jax.experimental.pallas import tpu_sc as plsc

DIM = 128
H = 4
HD = 32
BSZ = 1024
ACC_R = BSZ + 8
N_ROWS = 100000
N_PAD = 102400
BLK = 1000
NC = 2
NS = 16
NW = NC * NS
ROWS_W = N_PAD // NW
CH = 64
NCH = ROWS_W // CH
ROWS_T = BSZ // NS


def _bmat16():
    hrow = lax.broadcasted_iota(jnp.int32, (16, DIM), 0)
    hcol = lax.broadcasted_iota(jnp.int32, (16, DIM), 1) // HD
    return (hrow == hcol).astype(jnp.float32)


def _proj_body(batch_ref, x_ref, wvt_ref, a_ref, c_ref, bv_ref,
               ev_ref, s_ref, accs):
    i = pl.program_id(0)

    @pl.when(i == 0)
    def _init():
        accs[...] = jnp.zeros_like(accs)

    x = x_ref[...]
    v = jnp.dot(x, wvt_ref[...],
                preferred_element_type=jnp.float32) + bv_ref[...]
    attn = jnp.dot(x, a_ref[...],
                   preferred_element_type=jnp.float32) + c_ref[...]
    e = jnp.exp(attn)
    eb = jnp.dot(e, _bmat16(), preferred_element_type=jnp.float32)
    ev_ref[...] = eb * v

    brow = batch_ref[0]
    seg = lax.broadcasted_iota(jnp.int32, (BSZ, BLK), 0).astype(jnp.float32)
    oht = (jnp.broadcast_to(brow, (BSZ, BLK)) == seg).astype(jnp.bfloat16)
    accs[...] += jnp.dot(oht, e.astype(jnp.bfloat16),
                         preferred_element_type=jnp.float32)

    @pl.when(i == pl.num_programs(0) - 1)
    def _fin():
        s_ref[...] = accs[...]


def _sc_body(ev_h, idx_h, zp_h, pout_h, ibuf, vbuf_a, vbuf_b,
             accp, sem_a, sem_b):
    cid = lax.axis_index("c")
    sid = lax.axis_index("s")
    wid = cid * NS + sid

    @pl.when(sid == 0)
    def _init():
        pltpu.sync_copy(zp_h, accp)
    plsc.subcore_barrier()

    pltpu.sync_copy(idx_h.at[wid], ibuf)
    base = wid * ROWS_W

    def start_load(j, buf, sem):
        pltpu.async_copy(ev_h.at[pl.ds(base + j * CH, CH)], buf, sem)

    def wait_load(buf, sem):
        pltpu.make_async_copy(ev_h.at[pl.ds(0, CH)], buf, sem).wait()

    start_load(0, vbuf_a, sem_a)

    def outer(t, carry):
        jj = 2 * t
        start_load(jj + 1, vbuf_b, sem_b)
        wait_load(vbuf_a, sem_a)
        pltpu.sync_copy(vbuf_a, accp.at[ibuf.at[jj]], add=True)

        @pl.when(jj + 2 < NCH)
        def _():
            start_load(jj + 2, vbuf_a, sem_a)
        wait_load(vbuf_b, sem_b)
        pltpu.sync_copy(vbuf_b, accp.at[ibuf.at[jj + 1]], add=True)
        return carry

    lax.fori_loop(0, NCH // 2, outer, 0)
    plsc.subcore_barrier()

    r0 = sid * ROWS_T
    pltpu.sync_copy(accp.at[pl.ds(r0, ROWS_T)],
                    pout_h.at[cid, pl.ds(r0, ROWS_T)])


def _sc_call(ev, idx3):
    zp = jnp.zeros((ACC_R, DIM), jnp.float32)
    mesh = plsc.VectorSubcoreMesh(core_axis_name="c", subcore_axis_name="s")
    sc_fn = functools.partial(
        pl.kernel,
        out_type=jax.ShapeDtypeStruct((NC, BSZ, DIM), jnp.float32),
        mesh=mesh,
        scratch_types=[
            pltpu.VMEM((NCH, CH), jnp.int32),
            pltpu.VMEM((CH, DIM), jnp.float32),
            pltpu.VMEM((CH, DIM), jnp.float32),
            pltpu.VMEM_SHARED((ACC_R, DIM), jnp.float32),
            pltpu.SemaphoreType.DMA,
            pltpu.SemaphoreType.DMA,
        ],
    )(_sc_body)
    return sc_fn(ev, idx3, zp)


def _comb_body(p_ref, s_ref, out_ref):
    p = p_ref[0] + p_ref[1]
    sb = jnp.dot(s_ref[...], _bmat16(), preferred_element_type=jnp.float32)
    out_ref[...] = p / (sb + 1e-8)


def kernel(x, batch, query, W_k, b_k, W_v, b_v):
    scale = HD ** -0.5
    wkt = W_k.T
    a4 = scale * (wkt.reshape(DIM, H, HD) * query[None, :, :]).sum(-1)
    a16 = jnp.pad(a4, ((0, 0), (0, 12)))
    c4 = scale * (b_k.reshape(H, HD) * query).sum(-1)
    c16 = jnp.pad(c4, (0, 12)).reshape(1, 16)
    wvt = W_v.T
    bv = b_v.reshape(1, DIM)

    bi = batch.astype(jnp.int32)
    nblk = N_ROWS // BLK
    batchf = bi.astype(jnp.float32).reshape(nblk, 1, BLK)
    ev, s = pl.pallas_call(
        _proj_body,
        grid=(nblk,),
        in_specs=[
            pl.BlockSpec((1, 1, BLK), lambda i: (i, 0, 0)),
            pl.BlockSpec((BLK, DIM), lambda i: (i, 0)),
            pl.BlockSpec((DIM, DIM), lambda i: (0, 0)),
            pl.BlockSpec((DIM, 16), lambda i: (0, 0)),
            pl.BlockSpec((1, 16), lambda i: (0, 0)),
            pl.BlockSpec((1, DIM), lambda i: (0, 0)),
        ],
        out_specs=[
            pl.BlockSpec((BLK, DIM), lambda i: (i, 0)),
            pl.BlockSpec((BSZ, 16), lambda i: (0, 0)),
        ],
        out_shape=[
            jax.ShapeDtypeStruct((N_PAD, DIM), jnp.float32),
            jax.ShapeDtypeStruct((BSZ, 16), jnp.float32),
        ],
        scratch_shapes=[pltpu.VMEM((BSZ, 16), jnp.float32)],
    )(batchf, x, wvt, a16, c16, bv)

    idx3 = jnp.pad(bi, (0, N_PAD - N_ROWS),
                   constant_values=BSZ).reshape(NW, NCH, CH)
    p2 = _sc_call(ev, idx3)

    out = pl.pallas_call(
        _comb_body,
        out_shape=jax.ShapeDtypeStruct((BSZ, DIM), jnp.float32),
    )(p2, s)
    return out


# speedup vs baseline: 1.4130x; 1.0062x over previous
import functools
import jax
import jax.numpy as jnp
from jax import lax
from jax.experimental import pallas as pl
from jax.experimental.pallas import tpu as pltpu
from jax.experimental.pallas import tpu_sc as plsc

DIM = 128
H = 4
HD = 32
BSZ = 1024
ACC_R = BSZ + 8
N_ROWS = 100000
N_PAD = 102400
BLK = 1000
NC = 2
NS = 16
NW = NC * NS
ROWS_W = N_PAD // NW
CH = 64
NCH = ROWS_W // CH
ROWS_T = BSZ // NS


def _bmat16():
    hrow = lax.broadcasted_iota(jnp.int32, (16, DIM), 0)
    hcol = lax.broadcasted_iota(jnp.int32, (16, DIM), 1) // HD
    return (hrow == hcol).astype(jnp.float32)


def _proj_body(batch_ref, x_ref, wvt_ref, a_ref, c_ref, bv_ref,
               ev_ref, s_ref, accs):
    i = pl.program_id(0)

    @pl.when(i == 0)
    def _init():
        accs[...] = jnp.zeros_like(accs)

    x = x_ref[...]
    v = jnp.dot(x, wvt_ref[...],
                preferred_element_type=jnp.float32) + bv_ref[...]
    attn = jnp.dot(x, a_ref[...],
                   preferred_element_type=jnp.float32) + c_ref[...]
    e = jnp.exp(attn)
    eb = jnp.dot(e, _bmat16(), preferred_element_type=jnp.float32)
    ev_ref[...] = eb * v

    brow = batch_ref[0].astype(jnp.float32)
    seg = lax.broadcasted_iota(jnp.int32, (BSZ, BLK), 0).astype(jnp.float32)
    oht = (jnp.broadcast_to(brow, (BSZ, BLK)) == seg).astype(jnp.bfloat16)
    accs[...] += jnp.dot(oht, e.astype(jnp.bfloat16),
                         preferred_element_type=jnp.float32)

    @pl.when(i == pl.num_programs(0) - 1)
    def _fin():
        s_ref[...] = accs[...]


def _sc_body(ev_h, idx_h, pout_h, ibuf, vbuf_a, vbuf_b, zbuf,
             accp, sem_a, sem_b):
    cid = lax.axis_index("c")
    sid = lax.axis_index("s")
    wid = cid * NS + sid

    for r in range(8):
        for q in range(8):
            zbuf[r, pl.ds(q * 16, 16)] = jnp.zeros((16,), jnp.float32)
    for k in range(8):
        pltpu.sync_copy(zbuf, accp.at[pl.ds(sid * ROWS_T + k * 8, 8)])

    @pl.when(sid == 0)
    def _init_dump():
        pltpu.sync_copy(zbuf, accp.at[pl.ds(BSZ, 8)])
    plsc.subcore_barrier()

    pltpu.sync_copy(idx_h.at[wid], ibuf)
    base = wid * ROWS_W

    def start_load(j, buf, sem):
        pltpu.async_copy(ev_h.at[pl.ds(base + j * CH, CH)], buf, sem)

    def wait_load(buf, sem):
        pltpu.make_async_copy(ev_h.at[pl.ds(0, CH)], buf, sem).wait()

    start_load(0, vbuf_a, sem_a)

    def outer(t, carry):
        jj = 2 * t
        start_load(jj + 1, vbuf_b, sem_b)
        wait_load(vbuf_a, sem_a)
        pltpu.sync_copy(vbuf_a, accp.at[ibuf.at[jj]], add=True)

        @pl.when(jj + 2 < NCH)
        def _():
            start_load(jj + 2, vbuf_a, sem_a)
        wait_load(vbuf_b, sem_b)
        pltpu.sync_copy(vbuf_b, accp.at[ibuf.at[jj + 1]], add=True)
        return carry

    lax.fori_loop(0, NCH // 2, outer, 0)
    plsc.subcore_barrier()

    r0 = sid * ROWS_T
    pltpu.sync_copy(accp.at[pl.ds(r0, ROWS_T)],
                    pout_h.at[cid, pl.ds(r0, ROWS_T)])


def _sc_call(ev, idx3):
    mesh = plsc.VectorSubcoreMesh(core_axis_name="c", subcore_axis_name="s")
    sc_fn = functools.partial(
        pl.kernel,
        out_type=jax.ShapeDtypeStruct((NC, BSZ, DIM), jnp.float32),
        mesh=mesh,
        scratch_types=[
            pltpu.VMEM((NCH, CH), jnp.int32),
            pltpu.VMEM((CH, DIM), jnp.float32),
            pltpu.VMEM((CH, DIM), jnp.float32),
            pltpu.VMEM((8, DIM), jnp.float32),
            pltpu.VMEM_SHARED((ACC_R, DIM), jnp.float32),
            pltpu.SemaphoreType.DMA,
            pltpu.SemaphoreType.DMA,
        ],
    )(_sc_body)
    return sc_fn(ev, idx3)


def _comb_body(p_ref, s_ref, out_ref):
    p = p_ref[0] + p_ref[1]
    sb = jnp.dot(s_ref[...], _bmat16(), preferred_element_type=jnp.float32)
    out_ref[...] = p / (sb + 1e-8)


def kernel(x, batch, query, W_k, b_k, W_v, b_v):
    scale = HD ** -0.5
    wkt = W_k.T
    a4 = scale * (wkt.reshape(DIM, H, HD) * query[None, :, :]).sum(-1)
    a16 = jnp.pad(a4, ((0, 0), (0, 12)))
    c4 = scale * (b_k.reshape(H, HD) * query).sum(-1)
    c16 = jnp.pad(c4, (0, 12)).reshape(1, 16)
    wvt = W_v.T
    bv = b_v.reshape(1, DIM)

    bi = batch.astype(jnp.int32)
    nblk = N_ROWS // BLK
    batchf = bi.reshape(nblk, 1, BLK)
    ev, s = pl.pallas_call(
        _proj_body,
        grid=(nblk,),
        in_specs=[
            pl.BlockSpec((1, 1, BLK), lambda i: (i, 0, 0)),
            pl.BlockSpec((BLK, DIM), lambda i: (i, 0)),
            pl.BlockSpec((DIM, DIM), lambda i: (0, 0)),
            pl.BlockSpec((DIM, 16), lambda i: (0, 0)),
            pl.BlockSpec((1, 16), lambda i: (0, 0)),
            pl.BlockSpec((1, DIM), lambda i: (0, 0)),
        ],
        out_specs=[
            pl.BlockSpec((BLK, DIM), lambda i: (i, 0)),
            pl.BlockSpec((BSZ, 16), lambda i: (0, 0)),
        ],
        out_shape=[
            jax.ShapeDtypeStruct((N_PAD, DIM), jnp.float32),
            jax.ShapeDtypeStruct((BSZ, 16), jnp.float32),
        ],
        scratch_shapes=[pltpu.VMEM((BSZ, 16), jnp.float32)],
    )(batchf, x, wvt, a16, c16, bv)

    idx3 = jnp.pad(bi, (0, N_PAD - N_ROWS),
                   constant_values=BSZ).reshape(NW, NCH, CH)
    p2 = _sc_call(ev, idx3)

    out = pl.pallas_call(
        _comb_body,
        out_shape=jax.ShapeDtypeStruct((BSZ, DIM), jnp.float32),
    )(p2, s)
    return out
